# Initial kernel scaffold; baseline (speedup 1.0000x reference)
import jax
import jax.numpy as jnp
from jax.experimental import pallas as pl


def kernel(x, edge_index, W1, b1, W2, b2):
    raise NotImplementedError("write your pallas kernel here")


# speedup vs baseline: 8.9046x; 8.9046x over previous
import jax
import jax.numpy as jnp
from jax import lax
from jax.experimental import pallas as pl
from jax.experimental.pallas import tpu as pltpu
from jax.experimental.pallas import tpu_sc as plsc

N = 10000
NP = 10240
E = 320000
NW = 32
CH = 128
NCH = 80
EP = NW * NCH * CH
RPS = NP // 16
BM = 1024

_mesh = plsc.VectorSubcoreMesh(core_axis_name="c", subcore_axis_name="s")
_sc_params = pltpu.CompilerParams(use_tc_tiling_on_sc=False)


def _deg_body(src_hbm, dst_hbm, ones_hbm, zeros_hbm, dsrc_hbm, ddst_hbm,
              src_idx, dst_idx, ones_v, hist_src, hist_dst):
    cid = lax.axis_index("c")
    sid = lax.axis_index("s")
    wid = sid * 2 + cid
    pltpu.sync_copy(src_hbm.at[wid], src_idx)
    pltpu.sync_copy(dst_hbm.at[wid], dst_idx)
    pltpu.sync_copy(ones_hbm, ones_v)
    base = sid * RPS
    pltpu.sync_copy(zeros_hbm, hist_src.at[pl.ds(base, RPS)])
    pltpu.sync_copy(zeros_hbm, hist_dst.at[pl.ds(base, RPS)])
    plsc.subcore_barrier()

    @pl.loop(0, NCH)
    def _(j):
        pltpu.sync_copy(ones_v, hist_src.at[src_idx.at[j]], add=True)
        pltpu.sync_copy(ones_v, hist_dst.at[dst_idx.at[j]], add=True)

    plsc.subcore_barrier()
    out = cid * NP + base
    pltpu.sync_copy(hist_src.at[pl.ds(base, RPS)], dsrc_hbm.at[pl.ds(out, RPS)])
    pltpu.sync_copy(hist_dst.at[pl.ds(base, RPS)], ddst_hbm.at[pl.ds(out, RPS)])


def _degrees(src_p, dst_p, ones16, zeros16):
    fn = pl.kernel(
        _deg_body,
        out_type=[jax.ShapeDtypeStruct((2 * NP, 16), jnp.float32)] * 2,
        mesh=_mesh,
        scratch_types=[
            pltpu.VMEM((NCH, CH), jnp.int32),
            pltpu.VMEM((NCH, CH), jnp.int32),
            pltpu.VMEM((CH, 16), jnp.float32),
            pltpu.VMEM_SHARED((NP, 16), jnp.float32),
            pltpu.VMEM_SHARED((NP, 16), jnp.float32),
        ],
        compiler_params=_sc_params,
    )
    return fn(src_p, dst_p, ones16, zeros16)


def _make_scatter_body():
    D = 64

    def body(h_hbm, src_hbm, dst_hbm, zeros_hbm, out_hbm,
             src_idx, dst_idx, rows, acc, sem):
        cid = lax.axis_index("c")
        sid = lax.axis_index("s")
        wid = sid * 2 + cid
        pltpu.sync_copy(src_hbm.at[wid], src_idx)
        pltpu.sync_copy(dst_hbm.at[wid], dst_idx)
        base = sid * RPS
        pltpu.sync_copy(zeros_hbm, acc.at[pl.ds(base, RPS)])
        plsc.subcore_barrier()

        pltpu.async_copy(h_hbm.at[src_idx.at[0]], rows.at[0], sem).wait()

        @pl.loop(0, NCH - 1)
        def _(j):
            nxt = pltpu.async_copy(h_hbm.at[src_idx.at[j + 1]],
                                   rows.at[(j + 1) % 2], sem)
            pltpu.sync_copy(rows.at[j % 2], acc.at[dst_idx.at[j]], add=True)
            nxt.wait()

        pltpu.sync_copy(rows.at[(NCH - 1) % 2], acc.at[dst_idx.at[NCH - 1]],
                        add=True)
        plsc.subcore_barrier()
        pltpu.sync_copy(acc.at[pl.ds(base, RPS)],
                        out_hbm.at[pl.ds(cid * NP + base, RPS)])
    return body


def _scatter_pass(h, src_p, dst_p, zeros):
    fn = pl.kernel(
        _make_scatter_body(),
        out_type=jax.ShapeDtypeStruct((2 * NP, 64), jnp.float32),
        mesh=_mesh,
        scratch_types=[
            pltpu.VMEM((NCH, CH), jnp.int32),
            pltpu.VMEM((NCH, CH), jnp.int32),
            pltpu.VMEM((2, CH, 64), jnp.float32),
            pltpu.VMEM_SHARED((NP, 64), jnp.float32),
            pltpu.SemaphoreType.DMA,
        ],
        compiler_params=_sc_params,
    )
    return fn(h, src_p, dst_p, zeros)


def _mm_body(x_ref, w_ref, o_ref):
    o_ref[...] = jnp.dot(x_ref[...], w_ref[...],
                         preferred_element_type=jnp.float32)


def _matmul(x, w):
    k, m = w.shape
    return pl.pallas_call(
        _mm_body,
        grid=(NP // BM,),
        in_specs=[pl.BlockSpec((BM, k), lambda i: (i, 0)),
                  pl.BlockSpec((k, m), lambda i: (0, 0))],
        out_specs=pl.BlockSpec((BM, m), lambda i: (i, 0)),
        out_shape=jax.ShapeDtypeStruct((NP, m), jnp.float32),
    )(x, w)


def _norm(d_ref):
    deg = d_ref[0, :, 0:1] + d_ref[1, :, 0:1]
    return lax.rsqrt(jnp.maximum(deg, 1.0))


def _scale_body(p_ref, d_ref, lo_ref, hi_ref):
    h = p_ref[...] * _norm(d_ref)
    lo_ref[...] = h[:, :64]
    hi_ref[...] = h[:, 64:]


def _scale(p, dsrc):
    return pl.pallas_call(
        _scale_body,
        grid=(NP // BM,),
        in_specs=[pl.BlockSpec((BM, 128), lambda i: (i, 0)),
                  pl.BlockSpec((2, BM, 16), lambda i: (0, i, 0))],
        out_specs=[pl.BlockSpec((BM, 64), lambda i: (i, 0)),
                   pl.BlockSpec((BM, 64), lambda i: (i, 0))],
        out_shape=[jax.ShapeDtypeStruct((NP, 64), jnp.float32)] * 2,
    )(p, dsrc)


def _mid_body(plo_ref, phi_ref, dd_ref, ds_ref, b1_ref, w2_ref, o_ref):
    agg = jnp.concatenate([plo_ref[0] + plo_ref[1],
                           phi_ref[0] + phi_ref[1]], axis=1)
    h1 = jnp.maximum(agg * _norm(dd_ref) + b1_ref[...], 0.0)
    p2 = jnp.dot(h1, w2_ref[...], preferred_element_type=jnp.float32)
    o_ref[...] = p2 * _norm(ds_ref)


def _mid(parts_lo, parts_hi, ddst, dsrc, b1, W2):
    return pl.pallas_call(
        _mid_body,
        grid=(NP // BM,),
        in_specs=[pl.BlockSpec((2, BM, 64), lambda i: (0, i, 0)),
                  pl.BlockSpec((2, BM, 64), lambda i: (0, i, 0)),
                  pl.BlockSpec((2, BM, 16), lambda i: (0, i, 0)),
                  pl.BlockSpec((2, BM, 16), lambda i: (0, i, 0)),
                  pl.BlockSpec((1, 128), lambda i: (0, 0)),
                  pl.BlockSpec((128, 64), lambda i: (0, 0))],
        out_specs=pl.BlockSpec((BM, 64), lambda i: (i, 0)),
        out_shape=jax.ShapeDtypeStruct((NP, 64), jnp.float32),
    )(parts_lo, parts_hi, ddst, dsrc, b1, W2)


def _fin_body(pa_ref, dd_ref, b2_ref, o_ref):
    agg = pa_ref[0] + pa_ref[1]
    o_ref[...] = agg * _norm(dd_ref) + b2_ref[...]


def _fin(parts, ddst, b2):
    return pl.pallas_call(
        _fin_body,
        grid=(NP // BM,),
        in_specs=[pl.BlockSpec((2, BM, 64), lambda i: (0, i, 0)),
                  pl.BlockSpec((2, BM, 16), lambda i: (0, i, 0)),
                  pl.BlockSpec((1, 64), lambda i: (0, 0))],
        out_specs=pl.BlockSpec((BM, 64), lambda i: (i, 0)),
        out_shape=jax.ShapeDtypeStruct((NP, 64), jnp.float32),
    )(parts, ddst, b2)


def kernel(x, edge_index, W1, b1, W2, b2):
    src = edge_index[0]
    dst = edge_index[1]
    pad = N + jax.lax.rem(jnp.arange(EP - E, dtype=jnp.int32),
                          jnp.int32(NP - N))
    src_p = jnp.concatenate([src, pad]).reshape(NW, NCH, CH)
    dst_p = jnp.concatenate([dst, pad]).reshape(NW, NCH, CH)
    x_pad = jnp.pad(x, ((0, NP - N), (0, 0)))

    ones16 = jnp.ones((CH, 16), jnp.float32)
    zeros16 = jnp.zeros((RPS, 16), jnp.float32)
    zeros64 = jnp.zeros((RPS, 64), jnp.float32)

    dsrc, ddst = _degrees(src_p, dst_p, ones16, zeros16)
    dsrc = dsrc.reshape(2, NP, 16)
    ddst = ddst.reshape(2, NP, 16)

    p1 = _matmul(x_pad, W1)
    h_lo, h_hi = _scale(p1, dsrc)

    parts_lo = _scatter_pass(h_lo, src_p, dst_p, zeros64).reshape(2, NP, 64)
    parts_hi = _scatter_pass(h_hi, src_p, dst_p, zeros64).reshape(2, NP, 64)
    h2 = _mid(parts_lo, parts_hi, ddst, dsrc, b1.reshape(1, 128), W2)

    parts2 = _scatter_pass(h2, src_p, dst_p, zeros64).reshape(2, NP, 64)
    out = _fin(parts2, ddst, b2.reshape(1, 64))
    return out[:N]


# speedup vs baseline: 9.5389x; 1.0712x over previous
import jax
import jax.numpy as jnp
from jax import lax
from jax.experimental import pallas as pl
from jax.experimental.pallas import tpu as pltpu
from jax.experimental.pallas import tpu_sc as plsc

N = 10000
NP = 10240
E = 320000
NW = 32
CH = 128
NCH = 80
EP = NW * NCH * CH
RPS = NP // 16
BM = 1024

_mesh = plsc.VectorSubcoreMesh(core_axis_name="c", subcore_axis_name="s")
_sc_params = pltpu.CompilerParams(use_tc_tiling_on_sc=False,
                                  internal_scratch_in_bytes=0)


def _deg_body(src_hbm, dst_hbm, ones_hbm, zeros_hbm, dsrc_hbm, ddst_hbm,
              src_idx, dst_idx, ones_v, hist_src, hist_dst):
    cid = lax.axis_index("c")
    sid = lax.axis_index("s")
    wid = sid * 2 + cid
    pltpu.sync_copy(src_hbm.at[wid], src_idx)
    pltpu.sync_copy(dst_hbm.at[wid], dst_idx)
    pltpu.sync_copy(ones_hbm, ones_v)
    base = sid * RPS
    pltpu.sync_copy(zeros_hbm, hist_src.at[pl.ds(base, RPS)])
    pltpu.sync_copy(zeros_hbm, hist_dst.at[pl.ds(base, RPS)])
    plsc.subcore_barrier()

    @pl.loop(0, NCH)
    def _(j):
        pltpu.sync_copy(ones_v, hist_src.at[src_idx.at[j]], add=True)
        pltpu.sync_copy(ones_v, hist_dst.at[dst_idx.at[j]], add=True)

    plsc.subcore_barrier()
    out = cid * NP + base
    pltpu.sync_copy(hist_src.at[pl.ds(base, RPS)], dsrc_hbm.at[pl.ds(out, RPS)])
    pltpu.sync_copy(hist_dst.at[pl.ds(base, RPS)], ddst_hbm.at[pl.ds(out, RPS)])


def _degrees(src_p, dst_p, ones16, zeros16):
    fn = pl.kernel(
        _deg_body,
        out_type=[jax.ShapeDtypeStruct((2 * NP, 16), jnp.float32)] * 2,
        mesh=_mesh,
        scratch_types=[
            pltpu.VMEM((NCH, CH), jnp.int32),
            pltpu.VMEM((NCH, CH), jnp.int32),
            pltpu.VMEM((CH, 16), jnp.float32),
            pltpu.VMEM_SHARED((NP, 16), jnp.float32),
            pltpu.VMEM_SHARED((NP, 16), jnp.float32),
        ],
        compiler_params=_sc_params,
    )
    return fn(src_p, dst_p, ones16, zeros16)


def _make_scatter_body():
    D = 64

    def body(h_hbm, src_hbm, dst_hbm, zeros_hbm, out_hbm,
             src_idx, dst_idx, rows, operand, acc, sem):
        cid = lax.axis_index("c")
        sid = lax.axis_index("s")
        wid = sid * 2 + cid
        pltpu.sync_copy(src_hbm.at[wid], src_idx)
        pltpu.sync_copy(dst_hbm.at[wid], dst_idx)
        base = sid * RPS
        pltpu.sync_copy(zeros_hbm, acc.at[pl.ds(base, RPS)])
        pltpu.sync_copy(h_hbm.at[pl.ds(base, RPS)],
                        operand.at[pl.ds(base, RPS)])
        plsc.subcore_barrier()

        pltpu.async_copy(operand.at[src_idx.at[0]], rows.at[0], sem).wait()

        @pl.loop(0, NCH - 1)
        def _(j):
            nxt = pltpu.async_copy(operand.at[src_idx.at[j + 1]],
                                   rows.at[(j + 1) % 2], sem)
            pltpu.sync_copy(rows.at[j % 2], acc.at[dst_idx.at[j]], add=True)
            nxt.wait()

        pltpu.sync_copy(rows.at[(NCH - 1) % 2], acc.at[dst_idx.at[NCH - 1]],
                        add=True)
        plsc.subcore_barrier()
        pltpu.sync_copy(acc.at[pl.ds(base, RPS)],
                        out_hbm.at[pl.ds(cid * NP + base, RPS)])
    return body


def _scatter_pass(h, src_p, dst_p, zeros):
    fn = pl.kernel(
        _make_scatter_body(),
        out_type=jax.ShapeDtypeStruct((2 * NP, 64), jnp.float32),
        mesh=_mesh,
        scratch_types=[
            pltpu.VMEM((NCH, CH), jnp.int32),
            pltpu.VMEM((NCH, CH), jnp.int32),
            pltpu.VMEM((2, CH, 64), jnp.float32),
            pltpu.VMEM_SHARED((NP, 64), jnp.float32),
            pltpu.VMEM_SHARED((NP, 64), jnp.float32),
            pltpu.SemaphoreType.DMA,
        ],
        compiler_params=_sc_params,
    )
    return fn(h, src_p, dst_p, zeros)


def _mm_body(x_ref, w_ref, o_ref):
    o_ref[...] = jnp.dot(x_ref[...], w_ref[...],
                         preferred_element_type=jnp.float32)


def _matmul(x, w):
    k, m = w.shape
    return pl.pallas_call(
        _mm_body,
        grid=(NP // BM,),
        in_specs=[pl.BlockSpec((BM, k), lambda i: (i, 0)),
                  pl.BlockSpec((k, m), lambda i: (0, 0))],
        out_specs=pl.BlockSpec((BM, m), lambda i: (i, 0)),
        out_shape=jax.ShapeDtypeStruct((NP, m), jnp.float32),
    )(x, w)


def _norm(d_ref):
    deg = d_ref[0, :, 0:1] + d_ref[1, :, 0:1]
    return lax.rsqrt(jnp.maximum(deg, 1.0))


def _scale_body(p_ref, d_ref, lo_ref, hi_ref):
    h = p_ref[...] * _norm(d_ref)
    lo_ref[...] = h[:, :64]
    hi_ref[...] = h[:, 64:]


def _scale(p, dsrc):
    return pl.pallas_call(
        _scale_body,
        grid=(NP // BM,),
        in_specs=[pl.BlockSpec((BM, 128), lambda i: (i, 0)),
                  pl.BlockSpec((2, BM, 16), lambda i: (0, i, 0))],
        out_specs=[pl.BlockSpec((BM, 64), lambda i: (i, 0)),
                   pl.BlockSpec((BM, 64), lambda i: (i, 0))],
        out_shape=[jax.ShapeDtypeStruct((NP, 64), jnp.float32)] * 2,
    )(p, dsrc)


def _mid_body(plo_ref, phi_ref, dd_ref, ds_ref, b1_ref, w2_ref, o_ref):
    agg = jnp.concatenate([plo_ref[0] + plo_ref[1],
                           phi_ref[0] + phi_ref[1]], axis=1)
    h1 = jnp.maximum(agg * _norm(dd_ref) + b1_ref[...], 0.0)
    p2 = jnp.dot(h1, w2_ref[...], preferred_element_type=jnp.float32)
    o_ref[...] = p2 * _norm(ds_ref)


def _mid(parts_lo, parts_hi, ddst, dsrc, b1, W2):
    return pl.pallas_call(
        _mid_body,
        grid=(NP // BM,),
        in_specs=[pl.BlockSpec((2, BM, 64), lambda i: (0, i, 0)),
                  pl.BlockSpec((2, BM, 64), lambda i: (0, i, 0)),
                  pl.BlockSpec((2, BM, 16), lambda i: (0, i, 0)),
                  pl.BlockSpec((2, BM, 16), lambda i: (0, i, 0)),
                  pl.BlockSpec((1, 128), lambda i: (0, 0)),
                  pl.BlockSpec((128, 64), lambda i: (0, 0))],
        out_specs=pl.BlockSpec((BM, 64), lambda i: (i, 0)),
        out_shape=jax.ShapeDtypeStruct((NP, 64), jnp.float32),
    )(parts_lo, parts_hi, ddst, dsrc, b1, W2)


def _fin_body(pa_ref, dd_ref, b2_ref, o_ref):
    agg = pa_ref[0] + pa_ref[1]
    o_ref[...] = agg * _norm(dd_ref) + b2_ref[...]


def _fin(parts, ddst, b2):
    return pl.pallas_call(
        _fin_body,
        grid=(NP // BM,),
        in_specs=[pl.BlockSpec((2, BM, 64), lambda i: (0, i, 0)),
                  pl.BlockSpec((2, BM, 16), lambda i: (0, i, 0)),
                  pl.BlockSpec((1, 64), lambda i: (0, 0))],
        out_specs=pl.BlockSpec((BM, 64), lambda i: (i, 0)),
        out_shape=jax.ShapeDtypeStruct((NP, 64), jnp.float32),
    )(parts, ddst, b2)


def kernel(x, edge_index, W1, b1, W2, b2):
    src = edge_index[0]
    dst = edge_index[1]
    pad = N + jax.lax.rem(jnp.arange(EP - E, dtype=jnp.int32),
                          jnp.int32(NP - N))
    src_p = jnp.concatenate([src, pad]).reshape(NW, NCH, CH)
    dst_p = jnp.concatenate([dst, pad]).reshape(NW, NCH, CH)
    x_pad = jnp.pad(x, ((0, NP - N), (0, 0)))

    ones16 = jnp.ones((CH, 16), jnp.float32)
    zeros16 = jnp.zeros((RPS, 16), jnp.float32)
    zeros64 = jnp.zeros((RPS, 64), jnp.float32)

    dsrc, ddst = _degrees(src_p, dst_p, ones16, zeros16)
    dsrc = dsrc.reshape(2, NP, 16)
    ddst = ddst.reshape(2, NP, 16)

    p1 = _matmul(x_pad, W1)
    h_lo, h_hi = _scale(p1, dsrc)

    parts_lo = _scatter_pass(h_lo, src_p, dst_p, zeros64).reshape(2, NP, 64)
    parts_hi = _scatter_pass(h_hi, src_p, dst_p, zeros64).reshape(2, NP, 64)
    h2 = _mid(parts_lo, parts_hi, ddst, dsrc, b1.reshape(1, 128), W2)

    parts2 = _scatter_pass(h2, src_p, dst_p, zeros64).reshape(2, NP, 64)
    out = _fin(parts2, ddst, b2.reshape(1, 64))
    return out[:N]
